# Initial kernel scaffold; baseline (speedup 1.0000x reference)
import jax
import jax.numpy as jnp
from jax.experimental import pallas as pl


def kernel(x, t, static_table, dyn_xy, dyn_xz, dyn_yz):
    raise NotImplementedError("write your pallas kernel here")


# speedup vs baseline: 97.6644x; 97.6644x over previous
import numpy as np
import jax
import jax.numpy as jnp
from jax import lax
from jax.experimental import pallas as pl
from jax.experimental.pallas import tpu as pltpu
from jax.experimental.pallas import tpu_sc as plsc

_BASE_RES = 512.0
_MAX_RES = 32768.0
_NL = 8
_TIME_RES = 25
_SCALE = float(np.exp2(np.log2(_MAX_RES / _BASE_RES) / (_NL - 1)))
_RES = [np.float32(_BASE_RES * (_SCALE ** l) - 1.0) for l in range(_NL)]
_P1 = int(np.uint32(2654435761).view(np.int32))
_P2 = int(np.int32(805459861))
_TS = 1 << 19
_TXY = 1 << 14
_TXZ = 1 << 12
_OXZ = _NL * _TXY
_OYZ = _OXZ + _NL * _TXZ
_NDYN = _OYZ + _NL * _TXZ

_N = 262144
_NW = 32
_PT = _N // _NW
_C = 128
_G = _C // 16
_NCH = _PT // _C
_NU = 20


def _lerp_body(w_ref, a_ref, b_ref, o_ref):
    w = w_ref[0, 0]
    o_ref[...] = (1.0 - w) * a_ref[...] + w * b_ref[...]


def _lerp(a, b, w):
    rows = a.shape[0]
    blk = 768
    return pl.pallas_call(
        _lerp_body,
        out_shape=jax.ShapeDtypeStruct(a.shape, jnp.float32),
        grid=(rows // blk,),
        in_specs=[
            pl.BlockSpec(memory_space=pltpu.SMEM),
            pl.BlockSpec((blk, 128), lambda i: (i, 0)),
            pl.BlockSpec((blk, 128), lambda i: (i, 0)),
        ],
        out_specs=pl.BlockSpec((blk, 128), lambda i: (i, 0)),
    )(w, a, b)


def _sc_body(xT, st, dyn, resc, out_s, out_d,
             xb, resb, idxb, pb, wb, rows, osb, odb, sem):
    wid = lax.axis_index("s") * 2 + lax.axis_index("c")
    base = wid * _PT
    for j in range(3):
        pltpu.sync_copy(xT.at[pl.ds(j * _N + base, _PT)],
                        xb.at[pl.ds(j * _PT, _PT)])
    pltpu.sync_copy(resc, resb)
    lane = lax.iota(jnp.int32, 16)
    lane32 = lane * 32
    half = jnp.full((16,), 0.5, jnp.float32)
    one = jnp.full((16,), 1.0, jnp.float32)
    u_sp = [jnp.full((16,), u, jnp.int32) for u in range(_NU)]
    f_sp = [jnp.full((16,), f, jnp.int32) for f in range(4)]

    def chunk_body(ci, carry):
        co = ci * _C

        def level_body(l, carry2):
            vres = resb[pl.ds(l * 16, 16)]
            soff = l * (_TS // 2)
            oxy = l * (_TXY // 2)
            oxz = (_OXZ + l * _TXZ) // 2
            oyz = (_OYZ + l * _TXZ) // 2

            def grp(g, c3):
                g16 = g * 16
                o = co + g16
                x0 = xb[pl.ds(o, 16)]
                x1 = xb[pl.ds(_PT + o, 16)]
                x2 = xb[pl.ds(2 * _PT + o, 16)]
                pos0 = x0 * vres + half
                pos1 = x1 * vres + half
                pos2 = x2 * vres + half
                pi0 = pos0.astype(jnp.int32)
                pi1 = pos1.astype(jnp.int32)
                pi2 = pos2.astype(jnp.int32)
                w0 = pos0 - pi0.astype(jnp.float32)
                w1 = pos1 - pi1.astype(jnp.float32)
                w2 = pos2 - pi2.astype(jnp.float32)
                m1 = pi1 * _P1
                m2a = pi2 * _P2
                m2b = pi2 * _P1
                a0 = (pi0, pi0 + 1)
                a1 = (m1, m1 + _P1)
                a2s = (m2a, m2a + _P2)
                a2d = (m2b, m2b + _P1)
                ay = (pi1, pi1 + 1)
                ws = ((one - w0, w0), (one - w1, w1), (one - w2, w2))
                for c in range(8):
                    b0 = c & 1
                    b1 = (c >> 1) & 1
                    b2 = (c >> 2) & 1
                    h = (a0[b0] ^ a1[b1] ^ a2s[b2]) & (_TS - 1)
                    idxb[c, pl.ds(g16, 16)] = (h >> 1) + soff
                    pb[c, pl.ds(g16, 16)] = (h & 1) << 2
                    wb[pl.ds(c * _C + g16, 16)] = ws[0][b0] * ws[1][b1] * ws[2][b2]
                planes = (
                    (a0, a1, _TXY - 1, oxy, ws[0], ws[1], 8),
                    (a0, a2d, _TXZ - 1, oxz, ws[0], ws[2], 12),
                    (ay, a2d, _TXZ - 1, oyz, ws[1], ws[2], 16),
                )
                for A, B, mask, off, wA, wB, u0 in planes:
                    for c in range(4):
                        ba = c & 1
                        bb = (c >> 1) & 1
                        u = u0 + c
                        h = (A[ba] ^ B[bb]) & mask
                        idxb[u, pl.ds(g16, 16)] = (h >> 1) + off
                        pb[u, pl.ds(g16, 16)] = (h & 1) << 2
                        wb[pl.ds(u * _C + g16, 16)] = wA[ba] * wB[bb]
                return c3

            lax.fori_loop(0, _G, grp, 0)

            descs = []
            for u in range(8):
                descs.append(
                    pltpu.make_async_copy(st.at[idxb.at[u]], rows.at[u], sem))
            for u in range(8, _NU):
                descs.append(
                    pltpu.make_async_copy(dyn.at[idxb.at[u]], rows.at[u], sem))
            for dsc in descs:
                dsc.start()
            for dsc in descs:
                dsc.wait()

            def grp2(g, c3):
                g16 = g * 16
                pidx = lane + g16
                obase = lane32 + (g * 512 + 4 * l)
                accs = [jnp.zeros((16,), jnp.float32) for _ in range(4)]
                accp = [[jnp.zeros((16,), jnp.float32) for _ in range(4)]
                        for _ in range(3)]
                for u in range(_NU):
                    wc = wb[pl.ds(u * _C + g16, 16)]
                    pv = pb[u, pl.ds(g16, 16)]
                    tgt = accs if u < 8 else accp[(u - 8) // 4]
                    for f in range(4):
                        val = plsc.load_gather(rows, [u_sp[u], pidx, pv + f])
                        tgt[f] = tgt[f] + wc * val
                for f in range(4):
                    plsc.store_scatter(osb, [obase + f], accs[f])
                    plsc.store_scatter(odb, [obase + f],
                                       accp[0][f] * accp[1][f] * accp[2][f])
                return c3

            lax.fori_loop(0, _G, grp2, 0)
            return carry2

        lax.fori_loop(0, _NL, level_body, 0)
        pltpu.sync_copy(osb, out_s.at[pl.ds((base + co) * 32, _C * 32)])
        pltpu.sync_copy(odb, out_d.at[pl.ds((base + co) * 32, _C * 32)])
        return carry

    lax.fori_loop(0, _NCH, chunk_body, 0)


@jax.jit
def _hash_grid4d(x, t, static_table, dyn_xy, dyn_xz, dyn_yz):
    n = x.shape[0]
    tf = t.astype(jnp.float32)
    ti = tf * (_TIME_RES - 1)
    t0 = jnp.floor(ti)
    i1 = t0.astype(jnp.int32)
    i2 = jnp.ceil(ti).astype(jnp.int32)
    wt = (ti - t0).reshape(1, 1)

    def _slices(i):
        return jnp.concatenate([
            jnp.take(dyn_xy, i, axis=0).reshape(-1, 4),
            jnp.take(dyn_xz, i, axis=0).reshape(-1, 4),
            jnp.take(dyn_yz, i, axis=0).reshape(-1, 4),
        ], axis=0).reshape(-1, 128)

    dyneff = _lerp(_slices(i1), _slices(i2), wt).reshape(_NDYN // 2, 8)
    xT = x.T.reshape(-1)
    st = static_table.reshape(-1, 8)
    resc = jnp.asarray(np.repeat(np.asarray(_RES, np.float32), 16))

    mesh = plsc.VectorSubcoreMesh(
        core_axis_name="c", subcore_axis_name="s",
        num_cores=2, num_subcores=16)
    fs, fd = pl.kernel(
        _sc_body,
        out_type=[
            jax.ShapeDtypeStruct((_N * 32,), jnp.float32),
            jax.ShapeDtypeStruct((_N * 32,), jnp.float32),
        ],
        mesh=mesh,
        scratch_types=[
            pltpu.VMEM((3 * _PT,), jnp.float32),
            pltpu.VMEM((_NL * 16,), jnp.float32),
            pltpu.VMEM((_NU, _C), jnp.int32),
            pltpu.VMEM((_NU, _C), jnp.int32),
            pltpu.VMEM((_NU * _C,), jnp.float32),
            pltpu.VMEM((_NU, _C, 8), jnp.float32),
            pltpu.VMEM((_C * 32,), jnp.float32),
            pltpu.VMEM((_C * 32,), jnp.float32),
            pltpu.SemaphoreType.DMA,
        ],
        compiler_params=pltpu.CompilerParams(
            needs_layout_passes=False, use_tc_tiling_on_sc=False),
    )(xT, st, dyneff, resc)
    return fs.reshape(n, 32), fd.reshape(n, 32)


def kernel(x, t, static_table, dyn_xy, dyn_xz, dyn_yz):
    return _hash_grid4d(x, t, static_table, dyn_xy, dyn_xz, dyn_yz)


# speedup vs baseline: 97.8590x; 1.0020x over previous
import numpy as np
import jax
import jax.numpy as jnp
from jax import lax
from jax.experimental import pallas as pl
from jax.experimental.pallas import tpu as pltpu
from jax.experimental.pallas import tpu_sc as plsc

_BASE_RES = 512.0
_MAX_RES = 32768.0
_NL = 8
_TIME_RES = 25
_SCALE = float(np.exp2(np.log2(_MAX_RES / _BASE_RES) / (_NL - 1)))
_RES = [np.float32(_BASE_RES * (_SCALE ** l) - 1.0) for l in range(_NL)]
_P1 = int(np.uint32(2654435761).view(np.int32))
_P2 = int(np.int32(805459861))
_TS = 1 << 19
_TXY = 1 << 14
_TXZ = 1 << 12
_OXZ = _NL * _TXY
_OYZ = _OXZ + _NL * _TXZ
_NDYN = _OYZ + _NL * _TXZ

_N = 262144
_NW = 32
_PT = _N // _NW
_C = 128
_G = _C // 16
_NCH = _PT // _C
_NU = 20


def _lerp_body(w_ref, a_ref, b_ref, o_ref):
    w = w_ref[0, 0]
    o_ref[...] = (1.0 - w) * a_ref[...] + w * b_ref[...]


def _lerp(a, b, w):
    rows = a.shape[0]
    blk = 768
    return pl.pallas_call(
        _lerp_body,
        out_shape=jax.ShapeDtypeStruct(a.shape, jnp.float32),
        grid=(rows // blk,),
        in_specs=[
            pl.BlockSpec(memory_space=pltpu.SMEM),
            pl.BlockSpec((blk, 128), lambda i: (i, 0)),
            pl.BlockSpec((blk, 128), lambda i: (i, 0)),
        ],
        out_specs=pl.BlockSpec((blk, 128), lambda i: (i, 0)),
    )(w, a, b)


def _sc_body(xT, st, dyn, resc, out_s, out_d,
             xb, resb, idxs, idxd, pb, wb, rows_s, rows_d, osb, odb, sem):
    wid = lax.axis_index("s") * 2 + lax.axis_index("c")
    base = wid * _PT
    for j in range(3):
        pltpu.sync_copy(xT.at[pl.ds(j * _N + base, _PT)],
                        xb.at[pl.ds(j * _PT, _PT)])
    pltpu.sync_copy(resc, resb)
    lane = lax.iota(jnp.int32, 16)
    lane32 = lane * 32
    half = jnp.full((16,), 0.5, jnp.float32)
    one = jnp.full((16,), 1.0, jnp.float32)
    u_sp = [jnp.full((16,), u, jnp.int32) for u in range(_NU)]
    f_sp = [jnp.full((16,), f, jnp.int32) for f in range(4)]

    def chunk_body(ci, carry):
        co = ci * _C

        def level_body(l, carry2):
            vres = resb[pl.ds(l * 16, 16)]
            soff = l * (_TS // 2)
            oxy = l * (_TXY // 2)
            oxz = (_OXZ + l * _TXZ) // 2
            oyz = (_OYZ + l * _TXZ) // 2

            def grp(g, c3):
                g16 = g * 16
                o = co + g16
                x0 = xb[pl.ds(o, 16)]
                x1 = xb[pl.ds(_PT + o, 16)]
                x2 = xb[pl.ds(2 * _PT + o, 16)]
                pos0 = x0 * vres + half
                pos1 = x1 * vres + half
                pos2 = x2 * vres + half
                pi0 = pos0.astype(jnp.int32)
                pi1 = pos1.astype(jnp.int32)
                pi2 = pos2.astype(jnp.int32)
                w0 = pos0 - pi0.astype(jnp.float32)
                w1 = pos1 - pi1.astype(jnp.float32)
                w2 = pos2 - pi2.astype(jnp.float32)
                m1 = pi1 * _P1
                m2a = pi2 * _P2
                m2b = pi2 * _P1
                a0 = (pi0, pi0 + 1)
                a1 = (m1, m1 + _P1)
                a2s = (m2a, m2a + _P2)
                a2d = (m2b, m2b + _P1)
                ay = (pi1, pi1 + 1)
                ws = ((one - w0, w0), (one - w1, w1), (one - w2, w2))
                for c in range(8):
                    b0 = c & 1
                    b1 = (c >> 1) & 1
                    b2 = (c >> 2) & 1
                    h = (a0[b0] ^ a1[b1] ^ a2s[b2]) & (_TS - 1)
                    idxs[pl.ds(c * _C + g16, 16)] = (h >> 1) + soff
                    pb[c, pl.ds(g16, 16)] = (h & 1) << 2
                    wb[pl.ds(c * _C + g16, 16)] = ws[0][b0] * ws[1][b1] * ws[2][b2]
                planes = (
                    (a0, a1, _TXY - 1, oxy, ws[0], ws[1], 8),
                    (a0, a2d, _TXZ - 1, oxz, ws[0], ws[2], 12),
                    (ay, a2d, _TXZ - 1, oyz, ws[1], ws[2], 16),
                )
                for A, B, mask, off, wA, wB, u0 in planes:
                    for c in range(4):
                        ba = c & 1
                        bb = (c >> 1) & 1
                        u = u0 + c
                        h = (A[ba] ^ B[bb]) & mask
                        idxd[pl.ds((u - 8) * _C + g16, 16)] = (h >> 1) + off
                        pb[u, pl.ds(g16, 16)] = (h & 1) << 2
                        wb[pl.ds(u * _C + g16, 16)] = wA[ba] * wB[bb]
                return c3

            lax.fori_loop(0, _G, grp, 0)

            descs = [
                pltpu.make_async_copy(st.at[idxs], rows_s, sem),
                pltpu.make_async_copy(dyn.at[idxd], rows_d, sem),
            ]
            for dsc in descs:
                dsc.start()
            for dsc in descs:
                dsc.wait()

            def grp2(g, c3):
                g16 = g * 16
                pidx = lane + g16
                obase = lane32 + (g * 512 + 4 * l)
                accs = [jnp.zeros((16,), jnp.float32) for _ in range(4)]
                accp = [[jnp.zeros((16,), jnp.float32) for _ in range(4)]
                        for _ in range(3)]
                for u in range(_NU):
                    wc = wb[pl.ds(u * _C + g16, 16)]
                    pv = pb[u, pl.ds(g16, 16)]
                    if u < 8:
                        tgt, rbuf, ridx = accs, rows_s, pidx + u * _C
                    else:
                        tgt = accp[(u - 8) // 4]
                        rbuf, ridx = rows_d, pidx + (u - 8) * _C
                    for f in range(4):
                        val = plsc.load_gather(rbuf, [ridx, pv + f])
                        tgt[f] = tgt[f] + wc * val
                for f in range(4):
                    plsc.store_scatter(osb, [obase + f], accs[f])
                    plsc.store_scatter(odb, [obase + f],
                                       accp[0][f] * accp[1][f] * accp[2][f])
                return c3

            lax.fori_loop(0, _G, grp2, 0)
            return carry2

        lax.fori_loop(0, _NL, level_body, 0)
        pltpu.sync_copy(osb, out_s.at[pl.ds((base + co) * 32, _C * 32)])
        pltpu.sync_copy(odb, out_d.at[pl.ds((base + co) * 32, _C * 32)])
        return carry

    lax.fori_loop(0, _NCH, chunk_body, 0)


@jax.jit
def _hash_grid4d(x, t, static_table, dyn_xy, dyn_xz, dyn_yz):
    n = x.shape[0]
    tf = t.astype(jnp.float32)
    ti = tf * (_TIME_RES - 1)
    t0 = jnp.floor(ti)
    i1 = t0.astype(jnp.int32)
    i2 = jnp.ceil(ti).astype(jnp.int32)
    wt = (ti - t0).reshape(1, 1)

    def _slices(i):
        return jnp.concatenate([
            jnp.take(dyn_xy, i, axis=0).reshape(-1, 4),
            jnp.take(dyn_xz, i, axis=0).reshape(-1, 4),
            jnp.take(dyn_yz, i, axis=0).reshape(-1, 4),
        ], axis=0).reshape(-1, 128)

    dyneff = _lerp(_slices(i1), _slices(i2), wt).reshape(_NDYN // 2, 8)
    xT = x.T.reshape(-1)
    st = static_table.reshape(-1, 8)
    resc = jnp.asarray(np.repeat(np.asarray(_RES, np.float32), 16))

    mesh = plsc.VectorSubcoreMesh(
        core_axis_name="c", subcore_axis_name="s",
        num_cores=2, num_subcores=16)
    fs, fd = pl.kernel(
        _sc_body,
        out_type=[
            jax.ShapeDtypeStruct((_N * 32,), jnp.float32),
            jax.ShapeDtypeStruct((_N * 32,), jnp.float32),
        ],
        mesh=mesh,
        scratch_types=[
            pltpu.VMEM((3 * _PT,), jnp.float32),
            pltpu.VMEM((_NL * 16,), jnp.float32),
            pltpu.VMEM((8 * _C,), jnp.int32),
            pltpu.VMEM((12 * _C,), jnp.int32),
            pltpu.VMEM((_NU, _C), jnp.int32),
            pltpu.VMEM((_NU * _C,), jnp.float32),
            pltpu.VMEM((8 * _C, 8), jnp.float32),
            pltpu.VMEM((12 * _C, 8), jnp.float32),
            pltpu.VMEM((_C * 32,), jnp.float32),
            pltpu.VMEM((_C * 32,), jnp.float32),
            pltpu.SemaphoreType.DMA,
        ],
        compiler_params=pltpu.CompilerParams(
            needs_layout_passes=False, use_tc_tiling_on_sc=False),
    )(xT, st, dyneff, resc)
    return fs.reshape(n, 32), fd.reshape(n, 32)


def kernel(x, t, static_table, dyn_xy, dyn_xz, dyn_yz):
    return _hash_grid4d(x, t, static_table, dyn_xy, dyn_xz, dyn_yz)
